# baseline (speedup 1.0000x reference)
import functools

import jax
import jax.numpy as jnp
from jax import lax
from jax.experimental import pallas as pl
from jax.experimental.pallas import tpu as pltpu
from jax.experimental.pallas import tpu_sc as plsc

_V = 100000
_E = 64
_B = 16384
_L = 50
_PW = 16

_NC, _NS = 2, 16
_NW = _NC * _NS
_EPW = _B // _NW
_IPW = _EPW * _L
_IC = 100
_IROWS = _IPW // _IC
_CH_IR = 8
_CH_E = _CH_IR * _IC // _L
_NCH = _IROWS // _CH_IR

_VBLK = 2048
_VP = 49 * _VBLK
_DN = (((0,), (0,)), ((), ()))
_G = 128 // _PW
_OBLK0 = _VBLK // _G
_PROWS = _VP * _PW // 128


def _proj_body(wt_ref, w_ref, bias_ref, o_ref):
    wt = wt_ref[...]
    w = w_ref[...]
    m = jnp.concatenate(
        [lax.dot_general(wt[:, r * _OBLK0:(r + 1) * _OBLK0], w, _DN,
                         preferred_element_type=jnp.float32)
         for r in range(_G)], axis=1)
    o_ref[...] = m + bias_ref[...]


_proj = pl.pallas_call(
    _proj_body,
    grid=(_VP // _VBLK,),
    in_specs=[
        pl.BlockSpec((_E, _VBLK), lambda i: (0, i)),
        pl.BlockSpec((_E, _PW), lambda i: (0, 0)),
        pl.BlockSpec((1, 128), lambda i: (0, 0)),
    ],
    out_specs=pl.BlockSpec((_OBLK0, 128), lambda i: (i, 0)),
    out_shape=jax.ShapeDtypeStruct((_PROWS, 128), jnp.float32),
)


def _sc_body(idx_hbm, p_hbm, out_hbm, idx_v, r0, r1, out_v, sem0, sem1):
    wid = lax.axis_index("s") * _NC + lax.axis_index("c")
    irow0 = wid * _IROWS
    pltpu.sync_copy(idx_hbm.at[pl.ds(irow0, _IROWS)], idx_v)

    def fire(c, rows, sem):
        rbase = c * _CH_IR
        for j in range(_CH_IR):
            pltpu.async_copy(p_hbm.at[idx_v.at[rbase + j]],
                             rows.at[pl.ds(j * _IC, _IC)], sem)

    def drain(rows, sem):
        pltpu.make_async_copy(p_hbm.at[pl.ds(0, _CH_IR * _IC)], rows, sem).wait()

    def reduce(c, rows):
        ebase = c * _CH_E

        def ebody(e, carry):
            r = e * _L
            a = [rows[r + l, :] for l in range(4)]
            for l in range(4, _L):
                a[l % 4] = a[l % 4] + rows[r + l, :]
            out_v[ebase + e, :] = (a[0] + a[1]) + (a[2] + a[3])
            return carry

        lax.fori_loop(0, _CH_E, ebody, 0)

    fire(0, r0, sem0)

    def chunk_pair(c2, carry):
        c = c2 * 2
        fire(c + 1, r1, sem1)
        drain(r0, sem0)
        reduce(c, r0)

        @pl.when(c2 < _NCH // 2 - 1)
        def _():
            fire(c + 2, r0, sem0)

        drain(r1, sem1)
        reduce(c + 1, r1)
        return carry

    lax.fori_loop(0, _NCH // 2, chunk_pair, 0)
    pltpu.sync_copy(out_v, out_hbm.at[pl.ds(wid * _EPW, _EPW)])


_sc = functools.partial(
    pl.kernel,
    mesh=plsc.VectorSubcoreMesh(core_axis_name="c", subcore_axis_name="s"),
    out_type=jax.ShapeDtypeStruct((_B, _PW), jnp.float32),
    scratch_types=[
        pltpu.VMEM((_IROWS, _IC), jnp.int32),
        pltpu.VMEM((_CH_IR * _IC, _PW), jnp.float32),
        pltpu.VMEM((_CH_IR * _IC, _PW), jnp.float32),
        pltpu.VMEM((_EPW, _PW), jnp.float32),
        pltpu.SemaphoreType.DMA,
        pltpu.SemaphoreType.DMA,
    ],
    compiler_params=pltpu.CompilerParams(use_tc_tiling_on_sc=False),
)(_sc_body)


@jax.jit
def kernel(premise_indices, hypothesis_indices, W_prem, W_hypo, W_lin, b_lin):
    def perm(v):
        v = v.astype(jnp.int32)
        return (v & ~2047) | ((v & 255) << 3) | ((v >> 8) & 7)

    bpad = jnp.tile(
        jnp.zeros((1, _PW), jnp.float32).at[0, :3].set(b_lin / (2 * _L)),
        (1, 128 // _PW))

    pidx2 = perm(premise_indices).reshape(_B * _L // _IC, _IC)
    w1s = jnp.zeros((_E, _PW), jnp.float32).at[:, :3].set(W_lin[:, :_E].T / _L)
    p1 = _proj(W_prem.T, w1s, bpad).reshape(_VP, _PW)
    out1 = _sc(pidx2, p1)

    hidx2 = perm(hypothesis_indices).reshape(_B * _L // _IC, _IC)
    w2s = jnp.zeros((_E, _PW), jnp.float32).at[:, :3].set(W_lin[:, _E:].T / _L)
    p2 = _proj(W_hypo.T, w2s, bpad).reshape(_VP, _PW)
    out2 = _sc(hidx2, p2)

    return (out1 + out2)[:, :3]


# speedup vs baseline: 1.1179x; 1.1179x over previous
import functools

import jax
import jax.numpy as jnp
from jax import lax
from jax.experimental import pallas as pl
from jax.experimental.pallas import tpu as pltpu
from jax.experimental.pallas import tpu_sc as plsc

_V = 100000
_E = 64
_B = 16384
_L = 50
_PW = 16

_NC, _NS = 2, 16
_NW = _NC * _NS
_EPW = _B // _NW
_IPW = _EPW * _L
_IC = 100
_IROWS = _IPW // _IC
_CH_IR = 8
_CH_E = _CH_IR * _IC // _L
_NCH = _IROWS // _CH_IR

_VBLK = 4096
_VP = 25 * _VBLK
_DN = (((0,), (0,)), ((), ()))
_G = 128 // _PW
_OBLK0 = _VBLK // _G
_PROWS = _VP * _PW // 128


def _proj_body(wt_ref, w_ref, bias_ref, o_ref):
    wt = wt_ref[...]
    w = w_ref[...]
    m = jnp.concatenate(
        [lax.dot_general(wt[:, r * _OBLK0:(r + 1) * _OBLK0], w, _DN,
                         preferred_element_type=jnp.float32)
         for r in range(_G)], axis=1)
    o_ref[...] = m + bias_ref[...]


_proj = pl.pallas_call(
    _proj_body,
    grid=(_VP // _VBLK,),
    in_specs=[
        pl.BlockSpec((_E, _VBLK), lambda i: (0, i)),
        pl.BlockSpec((_E, _PW), lambda i: (0, 0)),
        pl.BlockSpec((1, 128), lambda i: (0, 0)),
    ],
    out_specs=pl.BlockSpec((_OBLK0, 128), lambda i: (i, 0)),
    out_shape=jax.ShapeDtypeStruct((_PROWS, 128), jnp.float32),
)


def _sc_body(idx_hbm, p_hbm, out_hbm, idx_v, r0, r1, out_v, sem0, sem1):
    wid = lax.axis_index("s") * _NC + lax.axis_index("c")
    irow0 = wid * _IROWS
    pltpu.sync_copy(idx_hbm.at[pl.ds(irow0, _IROWS)], idx_v)

    def fire(c, rows, sem):
        rbase = c * _CH_IR
        for j in range(_CH_IR):
            pltpu.async_copy(p_hbm.at[idx_v.at[rbase + j]],
                             rows.at[pl.ds(j * _IC, _IC)], sem)

    def drain(rows, sem):
        pltpu.make_async_copy(p_hbm.at[pl.ds(0, _CH_IR * _IC)], rows, sem).wait()

    def reduce(c, rows):
        ebase = c * _CH_E

        def ebody(e, carry):
            r = e * _L
            a = [rows[r + l, :] for l in range(4)]
            for l in range(4, _L):
                a[l % 4] = a[l % 4] + rows[r + l, :]
            out_v[ebase + e, :] = (a[0] + a[1]) + (a[2] + a[3])
            return carry

        lax.fori_loop(0, _CH_E, ebody, 0)

    fire(0, r0, sem0)

    def chunk_pair(c2, carry):
        c = c2 * 2
        fire(c + 1, r1, sem1)
        drain(r0, sem0)
        reduce(c, r0)

        @pl.when(c2 < _NCH // 2 - 1)
        def _():
            fire(c + 2, r0, sem0)

        drain(r1, sem1)
        reduce(c + 1, r1)
        return carry

    lax.fori_loop(0, _NCH // 2, chunk_pair, 0)
    pltpu.sync_copy(out_v, out_hbm.at[pl.ds(wid * _EPW, _EPW)])


_sc = functools.partial(
    pl.kernel,
    mesh=plsc.VectorSubcoreMesh(core_axis_name="c", subcore_axis_name="s"),
    out_type=jax.ShapeDtypeStruct((_B, _PW), jnp.float32),
    scratch_types=[
        pltpu.VMEM((_IROWS, _IC), jnp.int32),
        pltpu.VMEM((_CH_IR * _IC, _PW), jnp.float32),
        pltpu.VMEM((_CH_IR * _IC, _PW), jnp.float32),
        pltpu.VMEM((_EPW, _PW), jnp.float32),
        pltpu.SemaphoreType.DMA,
        pltpu.SemaphoreType.DMA,
    ],
    compiler_params=pltpu.CompilerParams(use_tc_tiling_on_sc=False),
)(_sc_body)


@jax.jit
def kernel(premise_indices, hypothesis_indices, W_prem, W_hypo, W_lin, b_lin):
    def perm(v):
        v = v.astype(jnp.int32)
        gshift = _OBLK0.bit_length() - 1
        return ((v & ~(_VBLK - 1))
                | ((v & (_OBLK0 - 1)) << 3)
                | ((v >> gshift) & (_G - 1)))

    bpad = jnp.tile(
        jnp.zeros((1, _PW), jnp.float32).at[0, :3].set(b_lin / (2 * _L)),
        (1, 128 // _PW))

    pidx2 = perm(premise_indices).reshape(_B * _L // _IC, _IC)
    w1s = jnp.zeros((_E, _PW), jnp.float32).at[:, :3].set(W_lin[:, :_E].T / _L)
    p1 = _proj(W_prem.T, w1s, bpad).reshape(_VP, _PW)
    out1 = _sc(pidx2, p1)

    hidx2 = perm(hypothesis_indices).reshape(_B * _L // _IC, _IC)
    w2s = jnp.zeros((_E, _PW), jnp.float32).at[:, :3].set(W_lin[:, _E:].T / _L)
    p2 = _proj(W_hypo.T, w2s, bpad).reshape(_VP, _PW)
    out2 = _sc(hidx2, p2)

    return (out1 + out2)[:, :3]


# speedup vs baseline: 1.1396x; 1.0194x over previous
import functools

import jax
import jax.numpy as jnp
from jax import lax
from jax.experimental import pallas as pl
from jax.experimental.pallas import tpu as pltpu
from jax.experimental.pallas import tpu_sc as plsc

_V = 100000
_E = 64
_B = 16384
_L = 50
_PW = 16

_NC, _NS = 2, 16
_NW = _NC * _NS
_EPW = _B // _NW
_CH_E = 16
_NCH = _EPW // _CH_E

_VBLK = 4096
_VP = 25 * _VBLK
_DN = (((0,), (0,)), ((), ()))
_G = 128 // _PW
_OBLK0 = _VBLK // _G
_PROWS = _VP * _PW // 128


def _proj_body(wt_ref, w_ref, bias_ref, o_ref):
    wt = wt_ref[...]
    w = w_ref[...]
    m = jnp.concatenate(
        [lax.dot_general(wt[:, r * _OBLK0:(r + 1) * _OBLK0], w, _DN,
                         preferred_element_type=jnp.float32)
         for r in range(_G)], axis=1)
    o_ref[...] = m + bias_ref[...]


_proj = pl.pallas_call(
    _proj_body,
    grid=(_VP // _VBLK,),
    in_specs=[
        pl.BlockSpec((_E, _VBLK), lambda i: (0, i)),
        pl.BlockSpec((_E, _PW), lambda i: (0, 0)),
        pl.BlockSpec((1, 128), lambda i: (0, 0)),
    ],
    out_specs=pl.BlockSpec((_OBLK0, 128), lambda i: (i, 0)),
    out_shape=jax.ShapeDtypeStruct((_PROWS, 128), jnp.float32),
)


def _make_sc(accumulate):
    def body(*refs):
        if accumulate:
            (idx_hbm, prev_hbm, p_hbm, out_hbm,
             idx_v, r0, r1, out_v, prev_v, sem0, sem1) = refs
        else:
            (idx_hbm, p_hbm, out_hbm,
             idx_v, r0, r1, out_v, sem0, sem1) = refs
        wid = lax.axis_index("s") * _NC + lax.axis_index("c")
        ebase0 = wid * _EPW
        if accumulate:
            prev_copy = pltpu.async_copy(
                prev_hbm.at[pl.ds(ebase0, _EPW)], prev_v, sem1)
        pltpu.sync_copy(idx_hbm.at[pl.ds(ebase0, _EPW)], idx_v)
        if accumulate:
            prev_copy.wait()

        def fire(c, rows, sem):
            eb = c * _CH_E
            for j in range(_CH_E):
                pltpu.async_copy(p_hbm.at[idx_v.at[eb + j]],
                                 rows.at[pl.ds(j * _L, _L)], sem)

        def drain(rows, sem):
            pltpu.make_async_copy(
                p_hbm.at[pl.ds(0, _CH_E * _L)], rows, sem).wait()

        def reduce(c, rows):
            ebase = c * _CH_E

            def ebody(e, carry):
                r = e * _L
                a = [rows[r + l, :] for l in range(4)]
                for l in range(4, _L):
                    a[l % 4] = a[l % 4] + rows[r + l, :]
                acc = (a[0] + a[1]) + (a[2] + a[3])
                if accumulate:
                    acc = acc + prev_v[ebase + e, :]
                out_v[ebase + e, :] = acc
                return carry

            lax.fori_loop(0, _CH_E, ebody, 0)

        fire(0, r0, sem0)

        def chunk_pair(c2, carry):
            c = c2 * 2
            fire(c + 1, r1, sem1)
            drain(r0, sem0)
            reduce(c, r0)

            @pl.when(c2 < _NCH // 2 - 1)
            def _():
                fire(c + 2, r0, sem0)

            drain(r1, sem1)
            reduce(c + 1, r1)
            return carry

        lax.fori_loop(0, _NCH // 2, chunk_pair, 0)
        pltpu.sync_copy(out_v, out_hbm.at[pl.ds(ebase0, _EPW)])

    scratch = [
        pltpu.VMEM((_EPW, _L), jnp.int32),
        pltpu.VMEM((_CH_E * _L, _PW), jnp.float32),
        pltpu.VMEM((_CH_E * _L, _PW), jnp.float32),
        pltpu.VMEM((_EPW, _PW), jnp.float32),
    ]
    if accumulate:
        scratch.append(pltpu.VMEM((_EPW, _PW), jnp.float32))
    scratch += [pltpu.SemaphoreType.DMA, pltpu.SemaphoreType.DMA]
    return functools.partial(
        pl.kernel,
        mesh=plsc.VectorSubcoreMesh(core_axis_name="c", subcore_axis_name="s"),
        out_type=jax.ShapeDtypeStruct((_B, _PW), jnp.float32),
        scratch_types=scratch,
        compiler_params=pltpu.CompilerParams(use_tc_tiling_on_sc=False),
    )(body)


_sc1 = _make_sc(False)
_sc2 = _make_sc(True)


@jax.jit
def kernel(premise_indices, hypothesis_indices, W_prem, W_hypo, W_lin, b_lin):
    def perm(v):
        v = v.astype(jnp.int32)
        gshift = _OBLK0.bit_length() - 1
        return ((v & ~(_VBLK - 1))
                | ((v & (_OBLK0 - 1)) << 3)
                | ((v >> gshift) & (_G - 1)))

    bpad = jnp.tile(
        jnp.zeros((1, _PW), jnp.float32).at[0, :3].set(b_lin / (2 * _L)),
        (1, 128 // _PW))

    pidx = perm(premise_indices)
    w1s = jnp.zeros((_E, _PW), jnp.float32).at[:, :3].set(W_lin[:, :_E].T / _L)
    p1 = _proj(W_prem.T, w1s, bpad).reshape(_VP, _PW)
    out1 = _sc1(pidx, p1)

    hidx = perm(hypothesis_indices)
    w2s = jnp.zeros((_E, _PW), jnp.float32).at[:, :3].set(W_lin[:, _E:].T / _L)
    p2 = _proj(W_hypo.T, w2s, bpad).reshape(_VP, _PW)
    out2 = _sc2(hidx, out1, p2)

    return out2[:, :3]


# speedup vs baseline: 1.3630x; 1.1961x over previous
import functools

import jax
import jax.numpy as jnp
from jax import lax
from jax.experimental import pallas as pl
from jax.experimental.pallas import tpu as pltpu
from jax.experimental.pallas import tpu_sc as plsc

_V = 100000
_E = 64
_B = 16384
_L = 50
_LP = 56
_PW = 16

_NC, _NS = 2, 16
_NW = _NC * _NS
_EPW = _B // _NW
_CE = 32
_NCH = _EPW // _CE

_VBLK = 4096
_VP = 25 * _VBLK
_DN = (((0,), (0,)), ((), ()))
_G = 128 // _PW
_OBLK0 = _VBLK // _G
_PROWS = _VP * _PW // 128

_FBLK = 1024


def _perm(v):
    gshift = _OBLK0.bit_length() - 1
    pshift = _G.bit_length() - 1
    return ((v & ~(_VBLK - 1))
            | ((v & (_OBLK0 - 1)) << pshift)
            | ((v >> gshift) & (_G - 1)))


def _fmt_body(pi_ref, hi_ref, po_ref, ho_ref):
    z = jnp.zeros((_LP - _L, _FBLK), jnp.int32)
    po_ref[...] = jnp.concatenate([_perm(pi_ref[...]), z], axis=0)
    ho_ref[...] = jnp.concatenate([_perm(hi_ref[...]), z], axis=0)


_fmt = pl.pallas_call(
    _fmt_body,
    grid=(_B // _FBLK,),
    in_specs=[
        pl.BlockSpec((_L, _FBLK), lambda i: (0, i)),
        pl.BlockSpec((_L, _FBLK), lambda i: (0, i)),
    ],
    out_specs=[
        pl.BlockSpec((_LP, _FBLK), lambda i: (0, i)),
        pl.BlockSpec((_LP, _FBLK), lambda i: (0, i)),
    ],
    out_shape=[
        jax.ShapeDtypeStruct((_LP, _B), jnp.int32),
        jax.ShapeDtypeStruct((_LP, _B), jnp.int32),
    ],
)


def _proj_body(wt_ref, w_ref, bias_ref, o_ref):
    wt = wt_ref[...]
    w = w_ref[...]
    m = jnp.concatenate(
        [lax.dot_general(wt[:, r * _OBLK0:(r + 1) * _OBLK0], w, _DN,
                         preferred_element_type=jnp.float32)
         for r in range(_G)], axis=1)
    o_ref[...] = m + bias_ref[...]


_proj = pl.pallas_call(
    _proj_body,
    grid=(_VP // _VBLK,),
    in_specs=[
        pl.BlockSpec((_E, _VBLK), lambda i: (0, i)),
        pl.BlockSpec((_E, _PW), lambda i: (0, 0)),
        pl.BlockSpec((1, 128), lambda i: (0, 0)),
    ],
    out_specs=pl.BlockSpec((_OBLK0, 128), lambda i: (i, 0)),
    out_shape=jax.ShapeDtypeStruct((_PROWS, 128), jnp.float32),
)


def _make_sc(accumulate):
    def body(*refs):
        if accumulate:
            (idx_hbm, prev_hbm, p_hbm, out_hbm,
             idx_v, r0, r1, out_v, prev_v, sem0, sem1) = refs
        else:
            (idx_hbm, p_hbm, out_hbm,
             idx_v, r0, r1, out_v, sem0, sem1) = refs
        wid = lax.axis_index("s") * _NC + lax.axis_index("c")
        ebase0 = wid * _EPW
        if accumulate:
            prev_copy = pltpu.async_copy(
                prev_hbm.at[pl.ds(ebase0, _EPW)], prev_v, sem1)
        pltpu.sync_copy(idx_hbm.at[pl.ds(0, _L), pl.ds(ebase0, _EPW)], idx_v)
        if accumulate:
            prev_copy.wait()

        def fire(c, rows, sem):
            e0 = c * _CE

            def lbody(l, carry):
                pltpu.async_copy(p_hbm.at[idx_v.at[l, pl.ds(e0, _CE)]],
                                 rows.at[pl.ds(l * _CE, _CE)], sem)
                return carry

            lax.fori_loop(0, _L, lbody, 0)

        def drain(rows, sem):
            pltpu.make_async_copy(
                p_hbm.at[pl.ds(0, _CE * _L)], rows, sem).wait()

        def reduce(c, rows):
            ebase = c * _CE

            def ebody(e, carry):
                a = [rows[e + l * _CE, :] for l in range(4)]
                for l in range(4, _L):
                    a[l % 4] = a[l % 4] + rows[e + l * _CE, :]
                acc = (a[0] + a[1]) + (a[2] + a[3])
                if accumulate:
                    acc = acc + prev_v[ebase + e, :]
                out_v[ebase + e, :] = acc
                return carry

            lax.fori_loop(0, _CE, ebody, 0)

        fire(0, r0, sem0)

        def chunk_pair(c2, carry):
            c = c2 * 2
            fire(c + 1, r1, sem1)
            drain(r0, sem0)
            reduce(c, r0)

            @pl.when(c2 < _NCH // 2 - 1)
            def _():
                fire(c + 2, r0, sem0)

            drain(r1, sem1)
            reduce(c + 1, r1)
            return carry

        lax.fori_loop(0, _NCH // 2, chunk_pair, 0)
        pltpu.sync_copy(out_v, out_hbm.at[pl.ds(ebase0, _EPW)])

    scratch = [
        pltpu.VMEM((_L, _EPW), jnp.int32),
        pltpu.VMEM((_CE * _L, _PW), jnp.float32),
        pltpu.VMEM((_CE * _L, _PW), jnp.float32),
        pltpu.VMEM((_EPW, _PW), jnp.float32),
    ]
    if accumulate:
        scratch.append(pltpu.VMEM((_EPW, _PW), jnp.float32))
    scratch += [pltpu.SemaphoreType.DMA, pltpu.SemaphoreType.DMA]
    return functools.partial(
        pl.kernel,
        mesh=plsc.VectorSubcoreMesh(core_axis_name="c", subcore_axis_name="s"),
        out_type=jax.ShapeDtypeStruct((_B, _PW), jnp.float32),
        scratch_types=scratch,
        compiler_params=pltpu.CompilerParams(use_tc_tiling_on_sc=False),
    )(body)


_sc1 = _make_sc(False)
_sc2 = _make_sc(True)


@jax.jit
def kernel(premise_indices, hypothesis_indices, W_prem, W_hypo, W_lin, b_lin):
    bpad = jnp.tile(
        jnp.zeros((1, _PW), jnp.float32).at[0, :3].set(b_lin / (2 * _L)),
        (1, 128 // _PW))

    pidx_f, hidx_f = _fmt(premise_indices.astype(jnp.int32).T,
                          hypothesis_indices.astype(jnp.int32).T)

    w1s = jnp.zeros((_E, _PW), jnp.float32).at[:, :3].set(W_lin[:, :_E].T / _L)
    p1 = _proj(W_prem.T, w1s, bpad).reshape(_VP, _PW)
    out1 = _sc1(pidx_f, p1)

    w2s = jnp.zeros((_E, _PW), jnp.float32).at[:, :3].set(W_lin[:, _E:].T / _L)
    p2 = _proj(W_hypo.T, w2s, bpad).reshape(_VP, _PW)
    out2 = _sc2(hidx_f, out1, p2)

    return out2[:, :3]


# speedup vs baseline: 1.3690x; 1.0044x over previous
import functools

import jax
import jax.numpy as jnp
from jax import lax
from jax.experimental import pallas as pl
from jax.experimental.pallas import tpu as pltpu
from jax.experimental.pallas import tpu_sc as plsc

_V = 100000
_E = 64
_B = 16384
_L = 50
_LP = 56
_PW = 16

_NC, _NS = 2, 16
_NW = _NC * _NS
_EPW = _B // _NW
_CE = 32
_NCH = _EPW // _CE

_VBLK = 4096
_VP = 25 * _VBLK
_DN = (((0,), (0,)), ((), ()))
_G = 128 // _PW
_OBLK0 = _VBLK // _G
_PROWS = _VP * _PW // 128

_FBLK = 1024


def _perm(v):
    gshift = _OBLK0.bit_length() - 1
    pshift = _G.bit_length() - 1
    return ((v & ~(_VBLK - 1))
            | ((v & (_OBLK0 - 1)) << pshift)
            | ((v >> gshift) & (_G - 1)))


def _proj_body(wt_ref, w_ref, bias_ref, o_ref):
    wt = wt_ref[...]
    w = w_ref[...]
    m = jnp.concatenate(
        [lax.dot_general(wt[:, r * _OBLK0:(r + 1) * _OBLK0], w, _DN,
                         preferred_element_type=jnp.float32)
         for r in range(_G)], axis=1)
    o_ref[...] = m + bias_ref[...]


_proj = pl.pallas_call(
    _proj_body,
    grid=(_VP // _VBLK,),
    in_specs=[
        pl.BlockSpec((_E, _VBLK), lambda i: (0, i)),
        pl.BlockSpec((_E, _PW), lambda i: (0, 0)),
        pl.BlockSpec((1, 128), lambda i: (0, 0)),
    ],
    out_specs=pl.BlockSpec((_OBLK0, 128), lambda i: (i, 0)),
    out_shape=jax.ShapeDtypeStruct((_PROWS, 128), jnp.float32),
)


def _make_sc(accumulate):
    def body(*refs):
        if accumulate:
            (idx_hbm, prev_hbm, p_hbm, out_hbm,
             idx_v, r0, r1, out_v, prev_v, sem0, sem1) = refs
        else:
            (idx_hbm, p_hbm, out_hbm,
             idx_v, r0, r1, out_v, sem0, sem1) = refs
        wid = lax.axis_index("s") * _NC + lax.axis_index("c")
        ebase0 = wid * _EPW
        if accumulate:
            prev_copy = pltpu.async_copy(
                prev_hbm.at[pl.ds(ebase0, _EPW)], prev_v, sem1)
        pltpu.sync_copy(idx_hbm.at[pl.ds(0, _L), pl.ds(ebase0, _EPW)], idx_v)
        if accumulate:
            prev_copy.wait()

        def transform(c):
            e0 = c * _CE

            def lbody(l, carry):
                for k in range(_CE // 16):
                    s = pl.ds(e0 + 16 * k, 16)
                    idx_v[l, s] = _perm(idx_v[l, s])
                return carry

            lax.fori_loop(0, _L, lbody, 0)

        def fire(c, rows, sem):
            e0 = c * _CE

            def lbody(l, carry):
                pltpu.async_copy(p_hbm.at[idx_v.at[l, pl.ds(e0, _CE)]],
                                 rows.at[pl.ds(l * _CE, _CE)], sem)
                return carry

            lax.fori_loop(0, _L, lbody, 0)

        def drain(rows, sem):
            pltpu.make_async_copy(
                p_hbm.at[pl.ds(0, _CE * _L)], rows, sem).wait()

        def reduce(c, rows):
            ebase = c * _CE

            def ebody(e, carry):
                a = [rows[e + l * _CE, :] for l in range(4)]
                for l in range(4, _L):
                    a[l % 4] = a[l % 4] + rows[e + l * _CE, :]
                acc = (a[0] + a[1]) + (a[2] + a[3])
                if accumulate:
                    acc = acc + prev_v[ebase + e, :]
                out_v[ebase + e, :] = acc
                return carry

            lax.fori_loop(0, _CE, ebody, 0)

        transform(0)
        fire(0, r0, sem0)

        def chunk_pair(c2, carry):
            c = c2 * 2
            transform(c + 1)
            fire(c + 1, r1, sem1)
            drain(r0, sem0)
            reduce(c, r0)

            @pl.when(c2 < _NCH // 2 - 1)
            def _():
                transform(c + 2)
                fire(c + 2, r0, sem0)

            drain(r1, sem1)
            reduce(c + 1, r1)
            return carry

        lax.fori_loop(0, _NCH // 2, chunk_pair, 0)
        pltpu.sync_copy(out_v, out_hbm.at[pl.ds(ebase0, _EPW)])

    scratch = [
        pltpu.VMEM((_L, _EPW), jnp.int32),
        pltpu.VMEM((_CE * _L, _PW), jnp.float32),
        pltpu.VMEM((_CE * _L, _PW), jnp.float32),
        pltpu.VMEM((_EPW, _PW), jnp.float32),
    ]
    if accumulate:
        scratch.append(pltpu.VMEM((_EPW, _PW), jnp.float32))
    scratch += [pltpu.SemaphoreType.DMA, pltpu.SemaphoreType.DMA]
    return functools.partial(
        pl.kernel,
        mesh=plsc.VectorSubcoreMesh(core_axis_name="c", subcore_axis_name="s"),
        out_type=jax.ShapeDtypeStruct((_B, _PW), jnp.float32),
        scratch_types=scratch,
        compiler_params=pltpu.CompilerParams(use_tc_tiling_on_sc=False),
    )(body)


_sc1 = _make_sc(False)
_sc2 = _make_sc(True)


@jax.jit
def kernel(premise_indices, hypothesis_indices, W_prem, W_hypo, W_lin, b_lin):
    bpad = jnp.tile(
        jnp.zeros((1, _PW), jnp.float32).at[0, :3].set(b_lin / (2 * _L)),
        (1, 128 // _PW))

    pidx_f = premise_indices.astype(jnp.int32).T
    hidx_f = hypothesis_indices.astype(jnp.int32).T

    w1s = jnp.zeros((_E, _PW), jnp.float32).at[:, :3].set(W_lin[:, :_E].T / _L)
    p1 = _proj(W_prem.T, w1s, bpad).reshape(_VP, _PW)
    out1 = _sc1(pidx_f, p1)

    w2s = jnp.zeros((_E, _PW), jnp.float32).at[:, :3].set(W_lin[:, _E:].T / _L)
    p2 = _proj(W_hypo.T, w2s, bpad).reshape(_VP, _PW)
    out2 = _sc2(hidx_f, out1, p2)

    return out2[:, :3]


# speedup vs baseline: 1.4577x; 1.0648x over previous
import functools

import jax
import jax.numpy as jnp
from jax import lax
from jax.experimental import pallas as pl
from jax.experimental.pallas import tpu as pltpu
from jax.experimental.pallas import tpu_sc as plsc

_V = 100000
_E = 64
_B = 16384
_L = 50
_LP = 56
_PW = 16

_NC, _NS = 2, 16
_NW = _NC * _NS
_EPW = _B // _NW
_CE = 32
_NCH = _EPW // _CE

_VBLK = 4096
_VP = 25 * _VBLK
_DN = (((0,), (0,)), ((), ()))
_G = 128 // _PW
_OBLK0 = _VBLK // _G
_PROWS = _VP * _PW // 128

_FBLK = 1024


def _perm(v):
    gshift = _OBLK0.bit_length() - 1
    pshift = _G.bit_length() - 1
    return ((v & ~(_VBLK - 1))
            | ((v & (_OBLK0 - 1)) << pshift)
            | ((v >> gshift) & (_G - 1)))


def _proj_body(wt_ref, w_ref, bias_ref, o_ref):
    wt = wt_ref[...]
    w = w_ref[...]
    m = jnp.concatenate(
        [lax.dot_general(wt[:, r * _OBLK0:(r + 1) * _OBLK0], w, _DN,
                         preferred_element_type=jnp.float32)
         for r in range(_G)], axis=1)
    o_ref[...] = m + bias_ref[...]


_proj = pl.pallas_call(
    _proj_body,
    grid=(_VP // _VBLK,),
    in_specs=[
        pl.BlockSpec((_E, _VBLK), lambda i: (0, i)),
        pl.BlockSpec((_E, _PW), lambda i: (0, 0)),
        pl.BlockSpec((1, 128), lambda i: (0, 0)),
    ],
    out_specs=pl.BlockSpec((_OBLK0, 128), lambda i: (i, 0)),
    out_shape=jax.ShapeDtypeStruct((_PROWS, 128), jnp.float32),
)


def _make_sc(accumulate):
    def body(*refs):
        if accumulate:
            (idx_hbm, prev_hbm, p_hbm, out_hbm,
             idx_v, r0, r1, out_v, prev_v, sem0, sem1) = refs
        else:
            (idx_hbm, p_hbm, out_hbm,
             idx_v, r0, r1, out_v, sem0, sem1) = refs
        wid = lax.axis_index("s") * _NC + lax.axis_index("c")
        ebase0 = wid * _EPW
        if accumulate:
            prev_copy = pltpu.async_copy(
                prev_hbm.at[pl.ds(ebase0, _EPW)], prev_v, sem1)
        pltpu.sync_copy(idx_hbm.at[pl.ds(0, _L), pl.ds(ebase0, _EPW)], idx_v)
        if accumulate:
            prev_copy.wait()

        def fire(c, rows, sem):
            e0 = c * _CE

            def lbody(l, carry):
                pltpu.async_copy(p_hbm.at[idx_v.at[l, pl.ds(e0, _CE)]],
                                 rows.at[pl.ds(l * _CE, _CE)], sem)
                return carry

            lax.fori_loop(0, _L, lbody, 0)

        def drain(rows, sem):
            pltpu.make_async_copy(
                p_hbm.at[pl.ds(0, _CE * _L)], rows, sem).wait()

        def reduce(c, rows):
            ebase = c * _CE

            def ebody(e, carry):
                a = [rows[e + l * _CE, :] for l in range(4)]
                for l in range(4, _L):
                    a[l % 4] = a[l % 4] + rows[e + l * _CE, :]
                acc = (a[0] + a[1]) + (a[2] + a[3])
                if accumulate:
                    acc = acc + prev_v[ebase + e, :]
                out_v[ebase + e, :] = acc
                return carry

            lax.fori_loop(0, _CE, ebody, 0)

        fire(0, r0, sem0)

        def chunk_pair(c2, carry):
            c = c2 * 2
            fire(c + 1, r1, sem1)
            drain(r0, sem0)
            reduce(c, r0)

            @pl.when(c2 < _NCH // 2 - 1)
            def _():
                fire(c + 2, r0, sem0)

            drain(r1, sem1)
            reduce(c + 1, r1)
            return carry

        lax.fori_loop(0, _NCH // 2, chunk_pair, 0)
        pltpu.sync_copy(out_v, out_hbm.at[pl.ds(ebase0, _EPW)])

    scratch = [
        pltpu.VMEM((_L, _EPW), jnp.int32),
        pltpu.VMEM((_CE * _L, _PW), jnp.float32),
        pltpu.VMEM((_CE * _L, _PW), jnp.float32),
        pltpu.VMEM((_EPW, _PW), jnp.float32),
    ]
    if accumulate:
        scratch.append(pltpu.VMEM((_EPW, _PW), jnp.float32))
    scratch += [pltpu.SemaphoreType.DMA, pltpu.SemaphoreType.DMA]
    return functools.partial(
        pl.kernel,
        mesh=plsc.VectorSubcoreMesh(core_axis_name="c", subcore_axis_name="s"),
        out_type=jax.ShapeDtypeStruct((_B, _PW), jnp.float32),
        scratch_types=scratch,
        compiler_params=pltpu.CompilerParams(use_tc_tiling_on_sc=False),
    )(body)


_sc1 = _make_sc(False)
_sc2 = _make_sc(True)


@jax.jit
def kernel(premise_indices, hypothesis_indices, W_prem, W_hypo, W_lin, b_lin):
    bpad = jnp.tile(
        jnp.zeros((1, _PW), jnp.float32).at[0, :3].set(b_lin / (2 * _L)),
        (1, 128 // _PW))

    pidx_f = _perm(premise_indices.astype(jnp.int32)).T
    hidx_f = _perm(hypothesis_indices.astype(jnp.int32)).T

    w1s = jnp.zeros((_E, _PW), jnp.float32).at[:, :3].set(W_lin[:, :_E].T / _L)
    p1 = _proj(W_prem.T, w1s, bpad).reshape(_VP, _PW)
    out1 = _sc1(pidx_f, p1)

    w2s = jnp.zeros((_E, _PW), jnp.float32).at[:, :3].set(W_lin[:, _E:].T / _L)
    p2 = _proj(W_hypo.T, w2s, bpad).reshape(_VP, _PW)
    out2 = _sc2(hidx_f, out1, p2)

    return out2[:, :3]
